# Initial kernel scaffold; baseline (speedup 1.0000x reference)
import jax
import jax.numpy as jnp
from jax.experimental import pallas as pl


def kernel(hidden_states, W_gate, Wg, Ag, Bg, Wu, Au, Bu, Wd, Ad, Bd):
    raise NotImplementedError("write your pallas kernel here")


# speedup vs baseline: 8.6623x; 8.6623x over previous
import functools

import jax
import jax.numpy as jnp
from jax import lax
from jax.experimental import pallas as pl

_TB = 256


def _silu(v):
    return v * jax.nn.sigmoid(v)


def _moe_body(x_ref, wgate_ref, wg_ref, agf_ref, bgf_ref, wu_ref, auf_ref,
              buf_ref, wd_ref, adf_ref, bdf_ref, out_ref, *, rank):
    f32 = jnp.float32
    bf16 = jnp.bfloat16
    x = x_ref[...]
    tb = x.shape[0]
    num_e = wgate_ref.shape[1]
    er = bgf_ref.shape[0]
    xb = x.astype(bf16)

    logits = jnp.dot(xb, wgate_ref[...], preferred_element_type=f32)
    cols = lax.broadcasted_iota(jnp.int32, (tb, num_e), 1)
    m1 = jnp.max(logits, axis=-1, keepdims=True)
    i1 = jnp.min(jnp.where(logits == m1, cols, num_e), axis=-1, keepdims=True)
    l2 = jnp.where(cols == i1, jnp.finfo(f32).min, logits)
    m2 = jnp.max(l2, axis=-1, keepdims=True)
    i2 = jnp.min(jnp.where(l2 == m2, cols, num_e), axis=-1, keepdims=True)
    w1 = jax.nn.sigmoid(m1 - m2)
    w2 = jax.nn.sigmoid(m2 - m1)

    gate_sh = jnp.dot(xb, wg_ref[...], preferred_element_type=f32)
    up_sh = jnp.dot(xb, wu_ref[...], preferred_element_type=f32)
    xag = jnp.dot(xb, agf_ref[...], preferred_element_type=f32)
    xau = jnp.dot(xb, auf_ref[...], preferred_element_type=f32)

    col_exp = lax.broadcasted_iota(jnp.int32, (tb, er), 1) // rank

    acts = []
    for ei in (i1, i2):
        mk = col_exp == ei
        zg = jnp.where(mk, xag, 0.0).astype(bf16)
        zu = jnp.where(mk, xau, 0.0).astype(bf16)
        g = gate_sh + jnp.dot(zg, bgf_ref[...], preferred_element_type=f32)
        u = up_sh + jnp.dot(zu, buf_ref[...], preferred_element_type=f32)
        acts.append(_silu(g) * u)
    act1, act2 = acts

    act_comb = (w1 * act1 + w2 * act2).astype(bf16)
    down = jnp.dot(act_comb, wd_ref[...], preferred_element_type=f32)

    zd1 = jnp.where(col_exp == i1,
                    jnp.dot(act1.astype(bf16), adf_ref[...],
                            preferred_element_type=f32), 0.0)
    zd2 = jnp.where(col_exp == i2,
                    jnp.dot(act2.astype(bf16), adf_ref[...],
                            preferred_element_type=f32), 0.0)
    zd = (w1 * zd1 + w2 * zd2).astype(bf16)
    out_ref[...] = down + jnp.dot(zd, bdf_ref[...], preferred_element_type=f32)


def kernel(hidden_states, W_gate, Wg, Ag, Bg, Wu, Au, Bu, Wd, Ad, Bd):
    b, s, d = hidden_states.shape
    t = b * s
    num_e, rank = Ag.shape[0], Ag.shape[2]
    f = Wg.shape[1]
    er = num_e * rank
    bf16 = jnp.bfloat16

    x = hidden_states.reshape(t, d)
    agf = jnp.transpose(Ag, (1, 0, 2)).reshape(d, er).astype(bf16)
    auf = jnp.transpose(Au, (1, 0, 2)).reshape(d, er).astype(bf16)
    adf = jnp.transpose(Ad, (1, 0, 2)).reshape(f, er).astype(bf16)
    bgf = Bg.reshape(er, f).astype(bf16)
    buf = Bu.reshape(er, f).astype(bf16)
    bdf = Bd.reshape(er, d).astype(bf16)

    tb = _TB
    grid = (t // tb,)
    full = lambda shape: pl.BlockSpec(shape, lambda i: (0, 0))

    out = pl.pallas_call(
        functools.partial(_moe_body, rank=rank),
        grid=grid,
        in_specs=[
            pl.BlockSpec((tb, d), lambda i: (i, 0)),
            full((d, num_e)),
            full((d, f)),
            full((d, er)),
            full((er, f)),
            full((d, f)),
            full((d, er)),
            full((er, f)),
            full((f, d)),
            full((f, er)),
            full((er, d)),
        ],
        out_specs=pl.BlockSpec((tb, d), lambda i: (i, 0)),
        out_shape=jax.ShapeDtypeStruct((t, d), jnp.float32),
    )(x, W_gate.astype(bf16), Wg.astype(bf16), agf, bgf, Wu.astype(bf16), auf, buf,
      Wd.astype(bf16), adf, bdf)
    return out.reshape(b, s, d)


# speedup vs baseline: 8.8746x; 1.0245x over previous
import functools

import jax
import jax.numpy as jnp
from jax import lax
from jax.experimental import pallas as pl

_TB = 512


def _silu(v):
    return v * jax.nn.sigmoid(v)


def _moe_body(x_ref, wgate_ref, wg_ref, agf_ref, bgf_ref, wu_ref, auf_ref,
              buf_ref, wd_ref, adf_ref, bdf_ref, out_ref, *, rank):
    f32 = jnp.float32
    bf16 = jnp.bfloat16
    x = x_ref[...]
    tb = x.shape[0]
    num_e = wgate_ref.shape[1]
    er = bgf_ref.shape[0]
    xb = x.astype(bf16)

    logits = jnp.dot(xb, wgate_ref[...], preferred_element_type=f32)
    cols = lax.broadcasted_iota(jnp.int32, (tb, num_e), 1)
    m1 = jnp.max(logits, axis=-1, keepdims=True)
    i1 = jnp.min(jnp.where(logits == m1, cols, num_e), axis=-1, keepdims=True)
    l2 = jnp.where(cols == i1, jnp.finfo(f32).min, logits)
    m2 = jnp.max(l2, axis=-1, keepdims=True)
    i2 = jnp.min(jnp.where(l2 == m2, cols, num_e), axis=-1, keepdims=True)
    w1 = jax.nn.sigmoid(m1 - m2)
    w2 = jax.nn.sigmoid(m2 - m1)

    gate_sh = jnp.dot(xb, wg_ref[...], preferred_element_type=f32)
    up_sh = jnp.dot(xb, wu_ref[...], preferred_element_type=f32)
    xag = jnp.dot(xb, agf_ref[...], preferred_element_type=f32)
    xau = jnp.dot(xb, auf_ref[...], preferred_element_type=f32)

    col_exp = lax.broadcasted_iota(jnp.int32, (tb, er), 1) // rank

    acts = []
    for ei in (i1, i2):
        mk = col_exp == ei
        zg = jnp.where(mk, xag, 0.0).astype(bf16)
        zu = jnp.where(mk, xau, 0.0).astype(bf16)
        g = gate_sh + jnp.dot(zg, bgf_ref[...], preferred_element_type=f32)
        u = up_sh + jnp.dot(zu, buf_ref[...], preferred_element_type=f32)
        acts.append(_silu(g) * u)
    act1, act2 = acts

    act_comb = (w1 * act1 + w2 * act2).astype(bf16)
    down = jnp.dot(act_comb, wd_ref[...], preferred_element_type=f32)

    zd1 = jnp.where(col_exp == i1,
                    jnp.dot(act1.astype(bf16), adf_ref[...],
                            preferred_element_type=f32), 0.0)
    zd2 = jnp.where(col_exp == i2,
                    jnp.dot(act2.astype(bf16), adf_ref[...],
                            preferred_element_type=f32), 0.0)
    zd = (w1 * zd1 + w2 * zd2).astype(bf16)
    out_ref[...] = down + jnp.dot(zd, bdf_ref[...], preferred_element_type=f32)


def kernel(hidden_states, W_gate, Wg, Ag, Bg, Wu, Au, Bu, Wd, Ad, Bd):
    b, s, d = hidden_states.shape
    t = b * s
    num_e, rank = Ag.shape[0], Ag.shape[2]
    f = Wg.shape[1]
    er = num_e * rank
    bf16 = jnp.bfloat16

    x = hidden_states.reshape(t, d)
    agf = jnp.transpose(Ag, (1, 0, 2)).reshape(d, er).astype(bf16)
    auf = jnp.transpose(Au, (1, 0, 2)).reshape(d, er).astype(bf16)
    adf = jnp.transpose(Ad, (1, 0, 2)).reshape(f, er).astype(bf16)
    bgf = Bg.reshape(er, f).astype(bf16)
    buf = Bu.reshape(er, f).astype(bf16)
    bdf = Bd.reshape(er, d).astype(bf16)

    tb = _TB
    grid = (t // tb,)
    full = lambda shape: pl.BlockSpec(shape, lambda i: (0, 0))

    out = pl.pallas_call(
        functools.partial(_moe_body, rank=rank),
        grid=grid,
        in_specs=[
            pl.BlockSpec((tb, d), lambda i: (i, 0)),
            full((d, num_e)),
            full((d, f)),
            full((d, er)),
            full((er, f)),
            full((d, f)),
            full((d, er)),
            full((er, f)),
            full((f, d)),
            full((f, er)),
            full((er, d)),
        ],
        out_specs=pl.BlockSpec((tb, d), lambda i: (i, 0)),
        out_shape=jax.ShapeDtypeStruct((t, d), jnp.float32),
    )(x, W_gate.astype(bf16), Wg.astype(bf16), agf, bgf, Wu.astype(bf16), auf, buf,
      Wd.astype(bf16), adf, bdf)
    return out.reshape(b, s, d)


# speedup vs baseline: 9.0353x; 1.0181x over previous
import functools

import jax
import jax.numpy as jnp
from jax import lax
from jax.experimental import pallas as pl

_TB = 512


def _silu(v):
    return v * jax.nn.sigmoid(v)


def _moe_body(x_ref, wgate_ref, wg_ref, agf_ref, bgf_ref, wu_ref, auf_ref,
              buf_ref, wd_ref, adf_ref, bdf_ref, out_ref, *, rank):
    f32 = jnp.float32
    bf16 = jnp.bfloat16
    x = x_ref[...]
    tb = x.shape[0]
    num_e = wgate_ref.shape[1]
    er = bgf_ref.shape[0]
    xb = x.astype(bf16)

    logits = jnp.dot(xb, wgate_ref[...], preferred_element_type=f32)
    cols = lax.broadcasted_iota(jnp.int32, (tb, num_e), 1)
    m1 = jnp.max(logits, axis=-1, keepdims=True)
    i1 = jnp.min(jnp.where(logits == m1, cols, num_e), axis=-1, keepdims=True)
    l2 = jnp.where(cols == i1, jnp.finfo(f32).min, logits)
    m2 = jnp.max(l2, axis=-1, keepdims=True)
    i2 = jnp.min(jnp.where(l2 == m2, cols, num_e), axis=-1, keepdims=True)
    w1 = jax.nn.sigmoid(m1 - m2)
    w2 = jax.nn.sigmoid(m2 - m1)

    gate_sh = jnp.dot(xb, wg_ref[...], preferred_element_type=f32)
    up_sh = jnp.dot(xb, wu_ref[...], preferred_element_type=f32)
    xag = jnp.dot(xb, agf_ref[...],
                  preferred_element_type=f32).astype(bf16)
    xau = jnp.dot(xb, auf_ref[...], preferred_element_type=f32).astype(bf16)

    col_exp = lax.broadcasted_iota(jnp.int32, (tb, er), 1) // rank
    zero16 = jnp.zeros((), bf16)

    acts = []
    for ei in (i1, i2):
        mk = col_exp == ei
        zg = jnp.where(mk, xag, zero16)
        zu = jnp.where(mk, xau, zero16)
        g = gate_sh + jnp.dot(zg, bgf_ref[...], preferred_element_type=f32)
        u = up_sh + jnp.dot(zu, buf_ref[...], preferred_element_type=f32)
        acts.append((_silu(g) * u).astype(bf16))
    act1, act2 = acts

    act_comb = (w1 * act1.astype(f32) + w2 * act2.astype(f32)).astype(bf16)
    down = jnp.dot(act_comb, wd_ref[...], preferred_element_type=f32)

    zd1 = jnp.where(col_exp == i1,
                    jnp.dot(act1, adf_ref[...],
                            preferred_element_type=f32).astype(bf16), zero16)
    zd2 = jnp.where(col_exp == i2,
                    jnp.dot(act2, adf_ref[...],
                            preferred_element_type=f32).astype(bf16), zero16)
    zd = (w1 * zd1.astype(f32) + w2 * zd2.astype(f32)).astype(bf16)
    out_ref[...] = down + jnp.dot(zd, bdf_ref[...], preferred_element_type=f32)


def kernel(hidden_states, W_gate, Wg, Ag, Bg, Wu, Au, Bu, Wd, Ad, Bd):
    b, s, d = hidden_states.shape
    t = b * s
    num_e, rank = Ag.shape[0], Ag.shape[2]
    f = Wg.shape[1]
    er = num_e * rank
    bf16 = jnp.bfloat16

    x = hidden_states.reshape(t, d)
    agf = jnp.transpose(Ag, (1, 0, 2)).reshape(d, er).astype(bf16)
    auf = jnp.transpose(Au, (1, 0, 2)).reshape(d, er).astype(bf16)
    adf = jnp.transpose(Ad, (1, 0, 2)).reshape(f, er).astype(bf16)
    bgf = Bg.reshape(er, f).astype(bf16)
    buf = Bu.reshape(er, f).astype(bf16)
    bdf = Bd.reshape(er, d).astype(bf16)

    tb = _TB
    grid = (t // tb,)
    full = lambda shape: pl.BlockSpec(shape, lambda i: (0, 0))

    out = pl.pallas_call(
        functools.partial(_moe_body, rank=rank),
        grid=grid,
        in_specs=[
            pl.BlockSpec((tb, d), lambda i: (i, 0)),
            full((d, num_e)),
            full((d, f)),
            full((d, er)),
            full((er, f)),
            full((d, f)),
            full((d, er)),
            full((er, f)),
            full((f, d)),
            full((f, er)),
            full((er, d)),
        ],
        out_specs=pl.BlockSpec((tb, d), lambda i: (i, 0)),
        out_shape=jax.ShapeDtypeStruct((t, d), jnp.float32),
    )(x, W_gate.astype(bf16), Wg.astype(bf16), agf, bgf, Wu.astype(bf16), auf, buf,
      Wd.astype(bf16), adf, bdf)
    return out.reshape(b, s, d)


# speedup vs baseline: 10.1212x; 1.1202x over previous
import functools

import jax
import jax.numpy as jnp
from jax import lax
from jax.experimental import pallas as pl
from jax.experimental.pallas import tpu as pltpu

_TB = 256


def _silu(v):
    return v * jax.nn.sigmoid(v)


def _moe_body(x_ref, wgate_ref, wg_ref, agf_ref, bgf_ref, wu_ref, auf_ref,
              buf_ref, wd_ref, adf_ref, bdf_ref, out_ref,
              wg_bf, wu_bf, wd_bf, bg_bf, bu_bf, bd_bf, *, rank):
    f32 = jnp.float32
    bf16 = jnp.bfloat16
    x = x_ref[...]
    tb = x.shape[0]
    num_e = wgate_ref.shape[1]
    er = bgf_ref.shape[0]
    xb = x.astype(bf16)

    @pl.when(pl.program_id(0) == 0)
    def _cast_weights():
        wg_bf[...] = wg_ref[...].astype(bf16)
        wu_bf[...] = wu_ref[...].astype(bf16)
        wd_bf[...] = wd_ref[...].astype(bf16)
        bg_bf[...] = bgf_ref[...].astype(bf16)
        bu_bf[...] = buf_ref[...].astype(bf16)
        bd_bf[...] = bdf_ref[...].astype(bf16)

    logits = jnp.dot(xb, wgate_ref[...], preferred_element_type=f32)
    cols = lax.broadcasted_iota(jnp.int32, (tb, num_e), 1)
    m1 = jnp.max(logits, axis=-1, keepdims=True)
    i1 = jnp.min(jnp.where(logits == m1, cols, num_e), axis=-1, keepdims=True)
    l2 = jnp.where(cols == i1, jnp.finfo(f32).min, logits)
    m2 = jnp.max(l2, axis=-1, keepdims=True)
    i2 = jnp.min(jnp.where(l2 == m2, cols, num_e), axis=-1, keepdims=True)
    w1 = jax.nn.sigmoid(m1 - m2)
    w2 = jax.nn.sigmoid(m2 - m1)

    gate_sh = jnp.dot(xb, wg_bf[...], preferred_element_type=f32)
    up_sh = jnp.dot(xb, wu_bf[...], preferred_element_type=f32)
    xag = jnp.dot(xb, agf_ref[...],
                  preferred_element_type=f32).astype(bf16)
    xau = jnp.dot(xb, auf_ref[...], preferred_element_type=f32).astype(bf16)

    col_exp = lax.broadcasted_iota(jnp.int32, (tb, er), 1) // rank
    zero16 = jnp.zeros((), bf16)
    mk1 = col_exp == i1
    mk2 = col_exp == i2

    zg = jnp.concatenate([jnp.where(mk1, xag, zero16),
                          jnp.where(mk2, xag, zero16)], axis=0)
    zu = jnp.concatenate([jnp.where(mk1, xau, zero16),
                          jnp.where(mk2, xau, zero16)], axis=0)
    dg = jnp.dot(zg, bg_bf[...], preferred_element_type=f32)
    du = jnp.dot(zu, bu_bf[...], preferred_element_type=f32)

    g1 = gate_sh + dg[:tb]
    g2 = gate_sh + dg[tb:]
    u1 = up_sh + du[:tb]
    u2 = up_sh + du[tb:]
    a1 = (w1 * (_silu(g1) * u1)).astype(bf16)
    a2 = (w2 * (_silu(g2) * u2)).astype(bf16)

    down = jnp.dot(a1 + a2, wd_bf[...], preferred_element_type=f32)

    aw = jnp.concatenate([a1, a2], axis=0)
    zd12 = jnp.dot(aw, adf_ref[...], preferred_element_type=f32)
    zd = (jnp.where(mk1, zd12[:tb], 0.0)
          + jnp.where(mk2, zd12[tb:], 0.0)).astype(bf16)
    out_ref[...] = down + jnp.dot(zd, bd_bf[...], preferred_element_type=f32)


def kernel(hidden_states, W_gate, Wg, Ag, Bg, Wu, Au, Bu, Wd, Ad, Bd):
    b, s, d = hidden_states.shape
    t = b * s
    num_e, rank = Ag.shape[0], Ag.shape[2]
    f = Wg.shape[1]
    er = num_e * rank
    bf16 = jnp.bfloat16

    x = hidden_states.reshape(t, d)
    agf = jnp.transpose(Ag, (1, 0, 2)).reshape(d, er).astype(bf16)
    auf = jnp.transpose(Au, (1, 0, 2)).reshape(d, er).astype(bf16)
    adf = jnp.transpose(Ad, (1, 0, 2)).reshape(f, er).astype(bf16)

    tb = _TB
    grid = (t // tb,)
    full = lambda shape: pl.BlockSpec(shape, lambda i: (0, 0))

    out = pl.pallas_call(
        functools.partial(_moe_body, rank=rank),
        grid=grid,
        in_specs=[
            pl.BlockSpec((tb, d), lambda i: (i, 0)),
            full((d, num_e)),
            full((d, f)),
            full((d, er)),
            full((er, f)),
            full((d, f)),
            full((d, er)),
            full((er, f)),
            full((f, d)),
            full((f, er)),
            full((er, d)),
        ],
        out_specs=pl.BlockSpec((tb, d), lambda i: (i, 0)),
        out_shape=jax.ShapeDtypeStruct((t, d), jnp.float32),
        scratch_shapes=[
            pltpu.VMEM((d, f), bf16),
            pltpu.VMEM((d, f), bf16),
            pltpu.VMEM((f, d), bf16),
            pltpu.VMEM((er, f), bf16),
            pltpu.VMEM((er, f), bf16),
            pltpu.VMEM((er, d), bf16),
        ],
    )(x, W_gate.astype(bf16), Wg, agf, Bg.reshape(er, f), Wu, auf,
      Bu.reshape(er, f), Wd, adf, Bd.reshape(er, d))
    return out.reshape(b, s, d)
